# baseline (speedup 1.0000x reference)
import functools

import jax
import jax.numpy as jnp
from jax import lax
from jax.experimental import pallas as pl
from jax.experimental.pallas import tpu as pltpu
from jax.experimental.pallas import tpu_sc as plsc

N = 100000
DIM = 128
ATOM_DIM = 6
MAX_DIS = 10
LANES = 16

CHUNK = 400
SUB = 80
NSUB = CHUNK // SUB
NCHUNKS = N // CHUNK
NWORKERS = 32
CHUNKS_PER_WORKER = (NCHUNKS + NWORKERS - 1) // NWORKERS

TCB = 2000


def _sc_body(d_hbm, emb_hbm, de_hbm,
             d_v, idx_v, out0_v, out1_v, gsem, sem0, sem1):
    wid = lax.axis_index("c") * 16 + lax.axis_index("s")

    out_bufs = (out0_v, out1_v)
    sems = (sem0, sem1)
    copies = [None, None]

    for t in range(CHUNKS_PER_WORKER):
        k = jnp.minimum(wid + t * NWORKERS, NCHUNKS - 1)
        base = k * CHUNK
        buf = t % 2
        out_v = out_bufs[buf]

        pltpu.sync_copy(d_hbm.at[pl.ds(base, CHUNK)],
                        d_v.at[pl.ds(0, CHUNK)])

        for g in range(CHUNK // LANES):
            dv = d_v[pl.ds(g * LANES, LANES)]
            dc = jnp.where(dv > 1000, MAX_DIS + 1, jnp.minimum(dv, MAX_DIS))
            idx_v[g // (SUB // LANES), pl.ds((g % (SUB // LANES)) * LANES, LANES)] = dc

        if copies[buf] is not None:
            copies[buf].wait()

        gathers = [
            pltpu.async_copy(emb_hbm.at[idx_v.at[j]],
                             out_v.at[pl.ds(j * SUB, SUB)], gsem)
            for j in range(NSUB)
        ]
        for g in gathers:
            g.wait()

        copies[buf] = pltpu.async_copy(
            out_v, de_hbm.at[pl.ds(base, CHUNK)], sems[buf])

    for c in copies:
        c.wait()


def _tc_body(x_ref, de_ref, w_ref, b_ref, out_ref):
    out_ref[...] = (
        jnp.dot(x_ref[...], w_ref[...], preferred_element_type=jnp.float32)
        + de_ref[...] + b_ref[...])


@jax.jit
def _run(x, d, W, b2, emb_d):
    mesh = plsc.VectorSubcoreMesh(core_axis_name="c", subcore_axis_name="s")
    sc_kern = functools.partial(
        pl.kernel,
        mesh=mesh,
        out_type=jax.ShapeDtypeStruct((N, DIM), jnp.float32),
        scratch_types=[
            pltpu.VMEM((CHUNK + LANES,), jnp.int32),
            pltpu.VMEM((NSUB, SUB), jnp.int32),
            pltpu.VMEM((CHUNK, DIM), jnp.float32),
            pltpu.VMEM((CHUNK, DIM), jnp.float32),
            pltpu.SemaphoreType.DMA,
            pltpu.SemaphoreType.DMA,
            pltpu.SemaphoreType.DMA,
        ],
    )(_sc_body)
    de = sc_kern(d, emb_d)

    return pl.pallas_call(
        _tc_body,
        grid=(N // TCB,),
        in_specs=[
            pl.BlockSpec((TCB, ATOM_DIM), lambda i: (i, 0)),
            pl.BlockSpec((TCB, DIM), lambda i: (i, 0)),
            pl.BlockSpec((ATOM_DIM, DIM), lambda i: (0, 0)),
            pl.BlockSpec((1, DIM), lambda i: (0, 0)),
        ],
        out_specs=pl.BlockSpec((TCB, DIM), lambda i: (i, 0)),
        out_shape=jax.ShapeDtypeStruct((N, DIM), jnp.float32),
    )(x, de, W, b2)


def kernel(x, d, W, b, emb_d):
    return _run(x, d, W, b.reshape(1, DIM), emb_d)


# speedup vs baseline: 15.8834x; 15.8834x over previous
import functools

import jax
import jax.numpy as jnp
from jax import lax
from jax.experimental import pallas as pl
from jax.experimental.pallas import tpu as pltpu
from jax.experimental.pallas import tpu_sc as plsc

N = 100000
DIM = 128
ATOM_DIM = 6
MAX_DIS = 10
LANES = 16
NSEG = DIM // LANES

CHUNK = 400
NCHUNKS = N // CHUNK
NWORKERS = 32
CHUNKS_PER_WORKER = (NCHUNKS + NWORKERS - 1) // NWORKERS

TCB = 2000


def _toff(dvec):
    d_i = dvec[0]
    dc = jnp.where(d_i > 1000, MAX_DIS + 1, jnp.minimum(d_i, MAX_DIS))
    return dc * DIM


def _sc_body(d_hbm, embf_hbm, de_hbm,
             d_v, t2f_v, out0_v, out1_v, sem0, sem1):
    wid = lax.axis_index("c") * 16 + lax.axis_index("s")

    pltpu.sync_copy(embf_hbm, t2f_v)

    out_bufs = (out0_v, out1_v)
    sems = (sem0, sem1)
    copies = [None, None]

    for t in range(CHUNKS_PER_WORKER):
        k = jnp.minimum(wid + t * NWORKERS, NCHUNKS - 1)
        base = k * CHUNK
        buf = t % 2
        out_v = out_bufs[buf]

        pltpu.sync_copy(d_hbm.at[pl.ds(base, CHUNK)],
                        d_v.at[pl.ds(0, CHUNK)])
        if copies[buf] is not None:
            copies[buf].wait()

        def row(i, toff, out_v=out_v):
            toff_next = _toff(d_v[pl.ds(i + 1, LANES)])
            segs = [t2f_v[pl.ds(toff + s * LANES, LANES)]
                    for s in range(NSEG)]
            for s in range(NSEG):
                out_v[i, pl.ds(s * LANES, LANES)] = segs[s]
            return toff_next

        lax.fori_loop(0, CHUNK, row, _toff(d_v[pl.ds(0, LANES)]))
        copies[buf] = pltpu.async_copy(
            out_v, de_hbm.at[pl.ds(base, CHUNK)], sems[buf])

    for c in copies:
        c.wait()


def _tc_body(x_ref, de_ref, w_ref, b_ref, out_ref):
    out_ref[...] = (
        jnp.dot(x_ref[...], w_ref[...], preferred_element_type=jnp.float32)
        + de_ref[...] + b_ref[...])


@jax.jit
def _run(x, d, W, b2, embf):
    mesh = plsc.VectorSubcoreMesh(core_axis_name="c", subcore_axis_name="s")
    sc_kern = functools.partial(
        pl.kernel,
        mesh=mesh,
        out_type=jax.ShapeDtypeStruct((N, DIM), jnp.float32),
        scratch_types=[
            pltpu.VMEM((CHUNK + LANES,), jnp.int32),
            pltpu.VMEM(((MAX_DIS + 2) * DIM,), jnp.float32),
            pltpu.VMEM((CHUNK, DIM), jnp.float32),
            pltpu.VMEM((CHUNK, DIM), jnp.float32),
            pltpu.SemaphoreType.DMA,
            pltpu.SemaphoreType.DMA,
        ],
    )(_sc_body)
    de = sc_kern(d, embf)

    return pl.pallas_call(
        _tc_body,
        grid=(N // TCB,),
        in_specs=[
            pl.BlockSpec((TCB, ATOM_DIM), lambda i: (i, 0)),
            pl.BlockSpec((TCB, DIM), lambda i: (i, 0)),
            pl.BlockSpec((ATOM_DIM, DIM), lambda i: (0, 0)),
            pl.BlockSpec((1, DIM), lambda i: (0, 0)),
        ],
        out_specs=pl.BlockSpec((TCB, DIM), lambda i: (i, 0)),
        out_shape=jax.ShapeDtypeStruct((N, DIM), jnp.float32),
    )(x, de, W, b2)


def kernel(x, d, W, b, emb_d):
    return _run(x, d, W, b.reshape(1, DIM), emb_d.reshape(-1))


# speedup vs baseline: 19.0106x; 1.1969x over previous
import functools

import jax
import jax.numpy as jnp
from jax import lax
from jax.experimental import pallas as pl
from jax.experimental.pallas import tpu as pltpu
from jax.experimental.pallas import tpu_sc as plsc

N = 100000
DIM = 128
ATOM_DIM = 6
MAX_DIS = 10
LANES = 16
NSEG = DIM // LANES

HALF = N // 2
CHUNK = 400
NCHUNKS = HALF // CHUNK
NWORKERS = 32
CHUNKS_PER_WORKER = (NCHUNKS + NWORKERS - 1) // NWORKERS

TCB = 5000
NBLK = HALF // TCB


def _toff(dvec):
    d_i = dvec[0]
    dc = jnp.where(d_i > 1000, MAX_DIS + 1, jnp.minimum(d_i, MAX_DIS))
    return dc * DIM


def _sc_body(d_hbm, embf_hbm, de_hbm,
             d_v, t2f_v, out0_v, out1_v, sem0, sem1):
    wid = lax.axis_index("c") * 16 + lax.axis_index("s")

    pltpu.sync_copy(embf_hbm, t2f_v)

    out_bufs = (out0_v, out1_v)
    sems = (sem0, sem1)
    copies = [None, None]

    for t in range(CHUNKS_PER_WORKER):
        k = jnp.minimum(wid + t * NWORKERS, NCHUNKS - 1)
        base = k * CHUNK
        buf = t % 2
        out_v = out_bufs[buf]

        pltpu.sync_copy(d_hbm.at[pl.ds(base, CHUNK)],
                        d_v.at[pl.ds(0, CHUNK)])
        if copies[buf] is not None:
            copies[buf].wait()

        def row(i, toff, out_v=out_v):
            toff_next = _toff(d_v[pl.ds(i + 1, LANES)])
            segs = [t2f_v[pl.ds(toff + s * LANES, LANES)]
                    for s in range(NSEG)]
            for s in range(NSEG):
                out_v[i, pl.ds(s * LANES, LANES)] = segs[s]
            return toff_next

        lax.fori_loop(0, CHUNK, row, _toff(d_v[pl.ds(0, LANES)]))
        copies[buf] = pltpu.async_copy(
            out_v, de_hbm.at[pl.ds(base, CHUNK)], sems[buf])

    for c in copies:
        c.wait()


def _tc_body(x_ref, de_ref, w_ref, b_ref, out_ref):
    out_ref[...] = (
        jnp.dot(x_ref[...], w_ref[...], preferred_element_type=jnp.float32)
        + de_ref[...] + b_ref[...])


def _tc_body_alias(x_ref, de_ref, w_ref, b_ref, prev_ref, out_ref):
    del prev_ref
    _tc_body(x_ref, de_ref, w_ref, b_ref, out_ref)


@jax.jit
def _run(x, d, W, b2, embf):
    mesh = plsc.VectorSubcoreMesh(core_axis_name="c", subcore_axis_name="s")
    sc_kern = functools.partial(
        pl.kernel,
        mesh=mesh,
        out_type=jax.ShapeDtypeStruct((HALF, DIM), jnp.float32),
        scratch_types=[
            pltpu.VMEM((CHUNK + LANES,), jnp.int32),
            pltpu.VMEM(((MAX_DIS + 2) * DIM,), jnp.float32),
            pltpu.VMEM((CHUNK, DIM), jnp.float32),
            pltpu.VMEM((CHUNK, DIM), jnp.float32),
            pltpu.SemaphoreType.DMA,
            pltpu.SemaphoreType.DMA,
        ],
    )(_sc_body)
    de_a = sc_kern(d[:HALF], embf)
    de_b = sc_kern(d[HALF:], embf)

    out_a = pl.pallas_call(
        _tc_body,
        grid=(NBLK,),
        in_specs=[
            pl.BlockSpec((TCB, ATOM_DIM), lambda i: (i, 0)),
            pl.BlockSpec((TCB, DIM), lambda i: (i, 0)),
            pl.BlockSpec((ATOM_DIM, DIM), lambda i: (0, 0)),
            pl.BlockSpec((1, DIM), lambda i: (0, 0)),
        ],
        out_specs=pl.BlockSpec((TCB, DIM), lambda i: (i, 0)),
        out_shape=jax.ShapeDtypeStruct((N, DIM), jnp.float32),
    )(x, de_a, W, b2)

    return pl.pallas_call(
        _tc_body_alias,
        grid=(NBLK,),
        in_specs=[
            pl.BlockSpec((TCB, ATOM_DIM), lambda i: (i + NBLK, 0)),
            pl.BlockSpec((TCB, DIM), lambda i: (i, 0)),
            pl.BlockSpec((ATOM_DIM, DIM), lambda i: (0, 0)),
            pl.BlockSpec((1, DIM), lambda i: (0, 0)),
            pl.BlockSpec(memory_space=pl.ANY),
        ],
        out_specs=pl.BlockSpec((TCB, DIM), lambda i: (i + NBLK, 0)),
        out_shape=jax.ShapeDtypeStruct((N, DIM), jnp.float32),
        input_output_aliases={4: 0},
    )(x, de_b, W, b2, out_a)


def kernel(x, d, W, b, emb_d):
    return _run(x, d, W, b.reshape(1, DIM), emb_d.reshape(-1))
